# baseline (speedup 1.0000x reference)
import functools

import jax
import jax.numpy as jnp
from jax import lax
from jax.experimental import pallas as pl
from jax.experimental.pallas import tpu as pltpu
from jax.experimental.pallas import tpu_sc as plsc

N = 10000
NP = 10240
E = 160000
NT = 16
NCORE = 2
EPT = 10368
EPB = 162
NBLK = 81
NCHUNK = 8
CW = 128
RPT = NP // NT
DROW = N

_mesh = plsc.VectorSubcoreMesh(core_axis_name="c", subcore_axis_name="s")
_sc_params = pltpu.CompilerParams(needs_layout_passes=False)
_f32 = jnp.float32
_i32 = jnp.int32


def _prep_body(zt, srcp, y0, dinv_o, s1_o, s2_o, r_o,
               deg_v, src_v, part_v, dinv_b, s1_b, s2_b, r_b, scl_v, buf_v,
               deg_sh, dinv_sh):
    cid = lax.axis_index("c")
    sid = lax.axis_index("s")
    wid = sid * NCORE + cid
    z16 = jnp.zeros((16,), _f32)
    ones = jnp.ones((16,), _f32)
    iota = lax.iota(_i32, 16)

    pltpu.sync_copy(srcp.at[sid], src_v)

    def _zero_deg(i, c):
        deg_v[pl.ds(16 * i, 16)] = z16
        return c
    lax.fori_loop(0, NP // 16, _zero_deg, 0)

    def _edges(bk, c):
        for j in range(8):
            idx = src_v[bk, pl.ds(16 * j, 16)]
            valid = (128 * bk + 16 * j + iota) < N
            plsc.addupdate_scatter(deg_v, [idx], ones, mask=valid)
        return c
    lax.fori_loop(0, NBLK, _edges, 0)

    pltpu.sync_copy(deg_v, deg_sh.at[sid])
    plsc.subcore_barrier()
    pltpu.sync_copy(deg_sh.at[:, pl.ds(sid * 640, 640)], part_v)

    def _red(q, c):
        acc = part_v[0, pl.ds(16 * q, 16)]
        for t in range(1, NT):
            acc = acc + part_v[t, pl.ds(16 * q, 16)]
        deg_v[pl.ds(16 * q, 16)] = acc
        return c
    lax.fori_loop(0, 40, _red, 0)

    for i in range(40):
        d = deg_v[pl.ds(16 * i, 16)]
        bits = plsc.bitcast(d, _i32)
        y = plsc.bitcast(jnp.int32(0x5F3759DF) - jnp.right_shift(bits, 1), _f32)
        for _ in range(3):
            y = y * (1.5 - 0.5 * d * y * y)
        dv = jnp.where(d > 0.5, y, 0.0)
        dinv_b[pl.ds(16 * i, 16)] = dv
        s1_b[pl.ds(16 * i, 16)] = -(dv * dv)
        s2_b[pl.ds(16 * i, 16)] = -dv
        r_b[pl.ds(16 * i, 16)] = d * dv
    pltpu.sync_copy(dinv_b, dinv_sh.at[pl.ds(sid * 640, 640)])

    @pl.when(cid == 0)
    def _():
        pltpu.sync_copy(dinv_b, dinv_o.at[pl.ds(sid * 640, 640)])
        pltpu.sync_copy(s1_b, s1_o.at[pl.ds(sid * 640, 640)])
        pltpu.sync_copy(s2_b, s2_o.at[pl.ds(sid * 640, 640)])
        pltpu.sync_copy(r_b, r_o.at[pl.ds(sid * 640, 640)])

    plsc.subcore_barrier()

    pltpu.sync_copy(dinv_sh.at[pl.ds(wid * 320, 320)], scl_v)
    for c in range(NCHUNK):
        base = c * NP + wid * 320
        pltpu.sync_copy(zt.at[pl.ds(base, 320), :], buf_v)

        def _scale(i, cr):
            dv = plsc.load_gather(scl_v, [jnp.broadcast_to(i, (16,))])
            for j in range(8):
                buf_v[i, pl.ds(16 * j, 16)] = buf_v[i, pl.ds(16 * j, 16)] * dv
            return cr
        lax.fori_loop(0, 320, _scale, 0)
        pltpu.sync_copy(buf_v, y0.at[pl.ds(base, 320), :])


_prep = functools.partial(
    pl.kernel,
    out_type=(
        jax.ShapeDtypeStruct((NCHUNK * NP, CW), _f32),
        jax.ShapeDtypeStruct((NP,), _f32),
        jax.ShapeDtypeStruct((NP,), _f32),
        jax.ShapeDtypeStruct((NP,), _f32),
        jax.ShapeDtypeStruct((NP,), _f32),
    ),
    mesh=_mesh,
    compiler_params=_sc_params,
    scratch_types=[
        pltpu.VMEM((NP,), _f32),
        pltpu.VMEM((NBLK, 128), _i32),
        pltpu.VMEM((NT, 640), _f32),
        pltpu.VMEM((640,), _f32),
        pltpu.VMEM((640,), _f32),
        pltpu.VMEM((640,), _f32),
        pltpu.VMEM((640,), _f32),
        pltpu.VMEM((320,), _f32),
        pltpu.VMEM((320, CW), _f32),
        pltpu.VMEM_SHARED((NT, NP), _f32),
        pltpu.VMEM_SHARED((NP,), _f32),
    ],
)(_prep_body)


def _spmm_body(y, srcp, dstp, sc, z,
               src_v, dst_v, scs_v, rows_v, acc_sh,
               g0, g1, g2, s0, s1, s2):
    cid = lax.axis_index("c")
    sid = lax.axis_index("s")
    gsem = (g0, g1, g2)
    ssem = (s0, s1, s2)
    z16 = jnp.zeros((16,), _f32)

    pltpu.sync_copy(srcp.at[sid], src_v)
    pltpu.sync_copy(dstp.at[sid], dst_v)
    pltpu.sync_copy(sc.at[pl.ds(sid * 640, 640)], scs_v)

    def _bump(delta):
        def _b(bk, cr):
            for j in range(8):
                src_v[bk, pl.ds(16 * j, 16)] = (
                    src_v[bk, pl.ds(16 * j, 16)] + delta)
            return cr
        lax.fori_loop(0, NBLK, _b, 0)

    _bump(cid * NP)

    for ci in range(NCHUNK // NCORE):
        c = 2 * ci + cid
        if ci > 0:
            _bump(2 * NP)

        def _zob(i, cr):
            for j in range(8):
                rows_v[0, i, pl.ds(16 * j, 16)] = z16
            return cr
        lax.fori_loop(0, 64, _zob, 0)
        for k in range(RPT // 64):
            pltpu.sync_copy(rows_v.at[0],
                            acc_sh.at[pl.ds(sid * RPT + 64 * k, 64), :])
        plsc.subcore_barrier()

        def _srcs(b):
            return src_v.at[jnp.right_shift(b, 1),
                            pl.ds(64 * jnp.bitwise_and(b, 1), 64)]

        def _dsts(b):
            return dst_v.at[jnp.right_shift(b, 1),
                            pl.ds(64 * jnp.bitwise_and(b, 1), 64)]

        pltpu.async_copy(y.at[_srcs(jnp.int32(0))], rows_v.at[0], gsem[0])
        pltpu.async_copy(y.at[_srcs(jnp.int32(1))], rows_v.at[1], gsem[1])

        def _ring(t, cr):
            for i in range(3):
                b = 3 * t + i
                pltpu.make_async_copy(y.at[pl.ds(0, 64), :], rows_v.at[i],
                                      gsem[i]).wait()
                pltpu.async_copy(rows_v.at[i], acc_sh.at[_dsts(b)], ssem[i],
                                 add=True)

                @pl.when(b > 0)
                def _():
                    pltpu.make_async_copy(
                        y.at[pl.ds(0, 64), :], rows_v.at[(i + 2) % 3],
                        ssem[(i + 2) % 3]).wait()

                @pl.when(b + 2 < EPB)
                def _():
                    pltpu.async_copy(y.at[_srcs(b + 2)],
                                     rows_v.at[(i + 2) % 3],
                                     gsem[(i + 2) % 3])
            return cr
        lax.fori_loop(0, EPB // 3, _ring, 0)
        pltpu.make_async_copy(y.at[pl.ds(0, 64), :], rows_v.at[2],
                              ssem[2]).wait()
        plsc.subcore_barrier()

        for k in range(RPT // 64):
            pltpu.sync_copy(acc_sh.at[pl.ds(sid * RPT + 64 * k, 64), :],
                            rows_v.at[0])

            def _scale(i, cr):
                q = 64 * k + i
                sv = plsc.load_gather(scs_v, [jnp.broadcast_to(q, (16,))])
                for j in range(8):
                    rows_v[0, i, pl.ds(16 * j, 16)] = (
                        rows_v[0, i, pl.ds(16 * j, 16)] * sv)
                return cr
            lax.fori_loop(0, 64, _scale, 0)
            pltpu.sync_copy(
                rows_v.at[0], z.at[pl.ds(c * NP + sid * RPT + 64 * k, 64), :])


_spmm = functools.partial(
    pl.kernel,
    out_type=jax.ShapeDtypeStruct((NCHUNK * NP, CW), _f32),
    mesh=_mesh,
    compiler_params=_sc_params,
    scratch_types=[
        pltpu.VMEM((NBLK, 128), _i32),
        pltpu.VMEM((NBLK, 128), _i32),
        pltpu.VMEM((640,), _f32),
        pltpu.VMEM((3, 64, CW), _f32),
        pltpu.VMEM_SHARED((NP, CW), _f32),
        pltpu.SemaphoreType.DMA,
        pltpu.SemaphoreType.DMA,
        pltpu.SemaphoreType.DMA,
        pltpu.SemaphoreType.DMA,
        pltpu.SemaphoreType.DMA,
        pltpu.SemaphoreType.DMA,
    ],
)(_spmm_body)


def _tc_body(x_ref, y1a_ref, y1b_ref, t2a_ref, t2b_ref, r_ref, w_ref, b_ref,
             o_ref):
    rr = r_ref[...]
    t0 = x_ref[0]
    w0 = w_ref[0]
    w1 = w_ref[1]
    w2 = w_ref[2]
    acc = jnp.dot(t0, w0, preferred_element_type=_f32)
    acc += jnp.dot(y1a_ref[0] * rr, w1[:CW, :], preferred_element_type=_f32)
    acc += jnp.dot(y1b_ref[0] * rr, w1[CW:, :], preferred_element_type=_f32)
    acc += jnp.dot(t2a_ref[0], w2[:CW, :], preferred_element_type=_f32)
    acc += jnp.dot(t2b_ref[0], w2[CW:, :], preferred_element_type=_f32)
    acc += b_ref[...]
    o_ref[0] = jnp.where(acc > 0.0, acc,
                         jnp.exp(jnp.minimum(acc, 0.0)) - 1.0)


def kernel(x, edge_index, W, b):
    Bb, Nn, Fi = x.shape
    Fo = W.shape[2]
    RB = 1000

    ei = edge_index.astype(jnp.int32)
    srcp = jnp.pad(ei[0].reshape(NT, N), ((0, 0), (0, EPT - N))
                   ).reshape(NT, NBLK, 128)
    dstp = jnp.pad(ei[1].reshape(NT, N), ((0, 0), (0, EPT - N)),
                   constant_values=DROW).reshape(NT, NBLK, 128)
    zt = x.reshape(Bb, N, 2, CW).transpose(0, 2, 1, 3).reshape(NCHUNK, N, CW)
    zt = jnp.pad(zt, ((0, 0), (0, NP - N), (0, 0))).reshape(NCHUNK * NP, CW)

    y0, dinv2, s12, s22, r2 = _prep(zt, srcp)
    z1 = _spmm(y0, srcp, dstp, s12)
    z2 = _spmm(z1, srcp, dstp, s22)

    Wm = jnp.stack([W[0] - W[2], W[1], 2.0 * W[2]])
    z1v = z1.reshape(NCHUNK, NP, CW)
    z2v = z2.reshape(NCHUNK, NP, CW)
    rv = r2.reshape(NP, 1)

    out = pl.pallas_call(
        _tc_body,
        grid=(Bb, N // RB),
        in_specs=[
            pl.BlockSpec((1, RB, Fi), lambda bb, nn: (bb, nn, 0)),
            pl.BlockSpec((1, RB, CW), lambda bb, nn: (2 * bb, nn, 0)),
            pl.BlockSpec((1, RB, CW), lambda bb, nn: (2 * bb + 1, nn, 0)),
            pl.BlockSpec((1, RB, CW), lambda bb, nn: (2 * bb, nn, 0)),
            pl.BlockSpec((1, RB, CW), lambda bb, nn: (2 * bb + 1, nn, 0)),
            pl.BlockSpec((RB, 1), lambda bb, nn: (nn, 0)),
            pl.BlockSpec((3, Fi, Fo), lambda bb, nn: (0, 0, 0)),
            pl.BlockSpec((1, Fo), lambda bb, nn: (0, 0)),
        ],
        out_specs=pl.BlockSpec((1, RB, Fo), lambda bb, nn: (bb, nn, 0)),
        out_shape=jax.ShapeDtypeStruct((Bb, N, Fo), _f32),
    )(x, z1v, z1v, z2v, z2v, rv, Wm, b.reshape(1, Fo))
    return out


# speedup vs baseline: 1.0022x; 1.0022x over previous
import functools

import jax
import jax.numpy as jnp
from jax import lax
from jax.experimental import pallas as pl
from jax.experimental.pallas import tpu as pltpu
from jax.experimental.pallas import tpu_sc as plsc

N = 10000
NP = 10240
E = 160000
NT = 16
NCORE = 2
EPT = 10368
EPB = 162
NBLK = 81
NCHUNK = 8
CW = 128
RPT = NP // NT
DROW = N

_mesh = plsc.VectorSubcoreMesh(core_axis_name="c", subcore_axis_name="s")
_sc_params = pltpu.CompilerParams(needs_layout_passes=False)
_f32 = jnp.float32
_i32 = jnp.int32


def _prep_body(zt, srcp, y0, dinv_o, s1_o, s2_o, r_o,
               deg_v, src_v, part_v, dinv_b, s1_b, s2_b, r_b, scl_v, buf_v,
               deg_sh, dinv_sh):
    cid = lax.axis_index("c")
    sid = lax.axis_index("s")
    wid = sid * NCORE + cid
    z16 = jnp.zeros((16,), _f32)
    ones = jnp.ones((16,), _f32)
    iota = lax.iota(_i32, 16)

    pltpu.sync_copy(srcp.at[sid], src_v)

    def _zero_deg(i, c):
        deg_v[pl.ds(16 * i, 16)] = z16
        return c
    lax.fori_loop(0, NP // 16, _zero_deg, 0)

    def _edges(bk, c):
        for j in range(8):
            idx = src_v[bk, pl.ds(16 * j, 16)]
            valid = (128 * bk + 16 * j + iota) < N
            plsc.addupdate_scatter(deg_v, [idx], ones, mask=valid)
        return c
    lax.fori_loop(0, NBLK, _edges, 0)

    pltpu.sync_copy(deg_v, deg_sh.at[sid])
    plsc.subcore_barrier()
    pltpu.sync_copy(deg_sh.at[:, pl.ds(sid * 640, 640)], part_v)

    def _red(q, c):
        acc = part_v[0, pl.ds(16 * q, 16)]
        for t in range(1, NT):
            acc = acc + part_v[t, pl.ds(16 * q, 16)]
        deg_v[pl.ds(16 * q, 16)] = acc
        return c
    lax.fori_loop(0, 40, _red, 0)

    for i in range(40):
        d = deg_v[pl.ds(16 * i, 16)]
        bits = plsc.bitcast(d, _i32)
        y = plsc.bitcast(jnp.int32(0x5F3759DF) - jnp.right_shift(bits, 1), _f32)
        for _ in range(3):
            y = y * (1.5 - 0.5 * d * y * y)
        dv = jnp.where(d > 0.5, y, 0.0)
        dinv_b[pl.ds(16 * i, 16)] = dv
        s1_b[pl.ds(16 * i, 16)] = -(dv * dv)
        s2_b[pl.ds(16 * i, 16)] = -dv
        r_b[pl.ds(16 * i, 16)] = d * dv
    pltpu.sync_copy(dinv_b, dinv_sh.at[pl.ds(sid * 640, 640)])

    @pl.when(cid == 0)
    def _():
        pltpu.sync_copy(dinv_b, dinv_o.at[pl.ds(sid * 640, 640)])
        pltpu.sync_copy(s1_b, s1_o.at[pl.ds(sid * 640, 640)])
        pltpu.sync_copy(s2_b, s2_o.at[pl.ds(sid * 640, 640)])
        pltpu.sync_copy(r_b, r_o.at[pl.ds(sid * 640, 640)])

    plsc.subcore_barrier()

    pltpu.sync_copy(dinv_sh.at[pl.ds(wid * 320, 320)], scl_v)
    for c in range(NCHUNK):
        base = c * NP + wid * 320
        pltpu.sync_copy(zt.at[pl.ds(base, 320), :], buf_v)

        def _scale(i, cr):
            dv = plsc.load_gather(scl_v, [jnp.broadcast_to(i, (16,))])
            for j in range(8):
                buf_v[i, pl.ds(16 * j, 16)] = buf_v[i, pl.ds(16 * j, 16)] * dv
            return cr
        lax.fori_loop(0, 320, _scale, 0)
        pltpu.sync_copy(buf_v, y0.at[pl.ds(base, 320), :])


_prep = functools.partial(
    pl.kernel,
    out_type=(
        jax.ShapeDtypeStruct((NCHUNK * NP, CW), _f32),
        jax.ShapeDtypeStruct((NP,), _f32),
        jax.ShapeDtypeStruct((NP,), _f32),
        jax.ShapeDtypeStruct((NP,), _f32),
        jax.ShapeDtypeStruct((NP,), _f32),
    ),
    mesh=_mesh,
    compiler_params=_sc_params,
    scratch_types=[
        pltpu.VMEM((NP,), _f32),
        pltpu.VMEM((NBLK, 128), _i32),
        pltpu.VMEM((NT, 640), _f32),
        pltpu.VMEM((640,), _f32),
        pltpu.VMEM((640,), _f32),
        pltpu.VMEM((640,), _f32),
        pltpu.VMEM((640,), _f32),
        pltpu.VMEM((320,), _f32),
        pltpu.VMEM((320, CW), _f32),
        pltpu.VMEM_SHARED((NT, NP), _f32),
        pltpu.VMEM_SHARED((NP,), _f32),
    ],
)(_prep_body)


def _spmm_body(y, srcp, dstp, sc, z,
               src_v, dst_v, scs_v, rows_v, acc_sh,
               g0, g1, g2, s0, s1, s2):
    cid = lax.axis_index("c")
    sid = lax.axis_index("s")
    gsem = (g0, g1, g2)
    ssem = (s0, s1, s2)
    z16 = jnp.zeros((16,), _f32)

    pltpu.sync_copy(srcp.at[sid], src_v)
    pltpu.sync_copy(dstp.at[sid], dst_v)
    pltpu.sync_copy(sc.at[pl.ds(sid * 640, 640)], scs_v)

    def _bump(delta):
        def _b(bk, cr):
            for j in range(8):
                src_v[bk, pl.ds(16 * j, 16)] = (
                    src_v[bk, pl.ds(16 * j, 16)] + delta)
            return cr
        lax.fori_loop(0, NBLK, _b, 0)

    _bump(cid * NP)

    for ci in range(NCHUNK // NCORE):
        c = 2 * ci + cid
        if ci > 0:
            _bump(2 * NP)

        def _zob(i, cr):
            for j in range(8):
                rows_v[0, i, pl.ds(16 * j, 16)] = z16
            return cr
        lax.fori_loop(0, 64, _zob, 0)
        for k in range(RPT // 64):
            pltpu.sync_copy(rows_v.at[0],
                            acc_sh.at[pl.ds(sid * RPT + 64 * k, 64), :])
        plsc.subcore_barrier()

        def _srcs(row, half):
            return src_v.at[row, pl.ds(64 * half, 64)]

        def _dsts(row, half):
            return dst_v.at[row, pl.ds(64 * half, 64)]

        pltpu.async_copy(y.at[_srcs(0, 0)], rows_v.at[0], gsem[0])
        pltpu.async_copy(y.at[_srcs(0, 1)], rows_v.at[1], gsem[1])

        def _ring(t, cr):
            for i in range(6):
                b = 6 * t + i
                sl = i % 3
                nsl = (i + 2) % 3
                pltpu.make_async_copy(y.at[pl.ds(0, 64), :], rows_v.at[sl],
                                      gsem[sl]).wait()
                pltpu.async_copy(rows_v.at[sl],
                                 acc_sh.at[_dsts(3 * t + i // 2, i % 2)],
                                 ssem[sl], add=True)

                @pl.when(b > 0)
                def _():
                    pltpu.make_async_copy(y.at[pl.ds(0, 64), :],
                                          rows_v.at[nsl], ssem[nsl]).wait()

                @pl.when(b + 2 < EPB)
                def _():
                    nb = i + 2
                    pltpu.async_copy(y.at[_srcs(3 * t + nb // 2, nb % 2)],
                                     rows_v.at[nsl], gsem[nsl])
            return cr
        lax.fori_loop(0, EPB // 6, _ring, 0)
        pltpu.make_async_copy(y.at[pl.ds(0, 64), :], rows_v.at[2],
                              ssem[2]).wait()
        plsc.subcore_barrier()

        for k in range(RPT // 64):
            pltpu.sync_copy(acc_sh.at[pl.ds(sid * RPT + 64 * k, 64), :],
                            rows_v.at[0])

            def _scale(i, cr):
                q = 64 * k + i
                sv = plsc.load_gather(scs_v, [jnp.broadcast_to(q, (16,))])
                for j in range(8):
                    rows_v[0, i, pl.ds(16 * j, 16)] = (
                        rows_v[0, i, pl.ds(16 * j, 16)] * sv)
                return cr
            lax.fori_loop(0, 64, _scale, 0)
            pltpu.sync_copy(
                rows_v.at[0], z.at[pl.ds(c * NP + sid * RPT + 64 * k, 64), :])


_spmm = functools.partial(
    pl.kernel,
    out_type=jax.ShapeDtypeStruct((NCHUNK * NP, CW), _f32),
    mesh=_mesh,
    compiler_params=_sc_params,
    scratch_types=[
        pltpu.VMEM((NBLK, 128), _i32),
        pltpu.VMEM((NBLK, 128), _i32),
        pltpu.VMEM((640,), _f32),
        pltpu.VMEM((3, 64, CW), _f32),
        pltpu.VMEM_SHARED((NP, CW), _f32),
        pltpu.SemaphoreType.DMA,
        pltpu.SemaphoreType.DMA,
        pltpu.SemaphoreType.DMA,
        pltpu.SemaphoreType.DMA,
        pltpu.SemaphoreType.DMA,
        pltpu.SemaphoreType.DMA,
    ],
)(_spmm_body)


def _tc_body(x_ref, y1a_ref, y1b_ref, t2a_ref, t2b_ref, r_ref, w_ref, b_ref,
             o_ref):
    rr = r_ref[...]
    t0 = x_ref[0]
    w0 = w_ref[0]
    w1 = w_ref[1]
    w2 = w_ref[2]
    acc = jnp.dot(t0, w0, preferred_element_type=_f32)
    acc += jnp.dot(y1a_ref[0] * rr, w1[:CW, :], preferred_element_type=_f32)
    acc += jnp.dot(y1b_ref[0] * rr, w1[CW:, :], preferred_element_type=_f32)
    acc += jnp.dot(t2a_ref[0], w2[:CW, :], preferred_element_type=_f32)
    acc += jnp.dot(t2b_ref[0], w2[CW:, :], preferred_element_type=_f32)
    acc += b_ref[...]
    o_ref[0] = jnp.where(acc > 0.0, acc,
                         jnp.exp(jnp.minimum(acc, 0.0)) - 1.0)


def kernel(x, edge_index, W, b):
    Bb, Nn, Fi = x.shape
    Fo = W.shape[2]
    RB = 1000

    ei = edge_index.astype(jnp.int32)
    srcp = jnp.pad(ei[0].reshape(NT, N), ((0, 0), (0, EPT - N))
                   ).reshape(NT, NBLK, 128)
    dstp = jnp.pad(ei[1].reshape(NT, N), ((0, 0), (0, EPT - N)),
                   constant_values=DROW).reshape(NT, NBLK, 128)
    zt = x.reshape(Bb, N, 2, CW).transpose(0, 2, 1, 3).reshape(NCHUNK, N, CW)
    zt = jnp.pad(zt, ((0, 0), (0, NP - N), (0, 0))).reshape(NCHUNK * NP, CW)

    y0, dinv2, s12, s22, r2 = _prep(zt, srcp)
    z1 = _spmm(y0, srcp, dstp, s12)
    z2 = _spmm(z1, srcp, dstp, s22)

    Wm = jnp.stack([W[0] - W[2], W[1], 2.0 * W[2]])
    z1v = z1.reshape(NCHUNK, NP, CW)
    z2v = z2.reshape(NCHUNK, NP, CW)
    rv = r2.reshape(NP, 1)

    out = pl.pallas_call(
        _tc_body,
        grid=(Bb, N // RB),
        in_specs=[
            pl.BlockSpec((1, RB, Fi), lambda bb, nn: (bb, nn, 0)),
            pl.BlockSpec((1, RB, CW), lambda bb, nn: (2 * bb, nn, 0)),
            pl.BlockSpec((1, RB, CW), lambda bb, nn: (2 * bb + 1, nn, 0)),
            pl.BlockSpec((1, RB, CW), lambda bb, nn: (2 * bb, nn, 0)),
            pl.BlockSpec((1, RB, CW), lambda bb, nn: (2 * bb + 1, nn, 0)),
            pl.BlockSpec((RB, 1), lambda bb, nn: (nn, 0)),
            pl.BlockSpec((3, Fi, Fo), lambda bb, nn: (0, 0, 0)),
            pl.BlockSpec((1, Fo), lambda bb, nn: (0, 0)),
        ],
        out_specs=pl.BlockSpec((1, RB, Fo), lambda bb, nn: (bb, nn, 0)),
        out_shape=jax.ShapeDtypeStruct((Bb, N, Fo), _f32),
    )(x, z1v, z1v, z2v, z2v, rv, Wm, b.reshape(1, Fo))
    return out


# speedup vs baseline: 1.5990x; 1.5955x over previous
import functools

import jax
import jax.numpy as jnp
from jax import lax
from jax.experimental import pallas as pl
from jax.experimental.pallas import tpu as pltpu
from jax.experimental.pallas import tpu_sc as plsc

N = 10000
NP = 10240
E = 160000
NT = 16
NCORE = 2
EPT = 10112
EPB = 158
NBLK = 79
NCHUNK = 8
CW = 128
RPT = NP // NT
DROW = N

_mesh = plsc.VectorSubcoreMesh(core_axis_name="c", subcore_axis_name="s")
_sc_params = pltpu.CompilerParams(needs_layout_passes=False)
_f32 = jnp.float32
_i32 = jnp.int32


def _prep_body(zt, srcp, y0, dinv_o, s1_o, s2_o, r_o,
               deg_v, src_v, part_v, dinv_b, s1_b, s2_b, r_b, scl_v, buf_v,
               deg_sh, dinv_sh):
    cid = lax.axis_index("c")
    sid = lax.axis_index("s")
    wid = sid * NCORE + cid
    z16 = jnp.zeros((16,), _f32)
    ones = jnp.ones((16,), _f32)
    iota = lax.iota(_i32, 16)

    pltpu.sync_copy(srcp.at[sid], src_v)

    def _zero_deg(i, c):
        deg_v[pl.ds(16 * i, 16)] = z16
        return c
    lax.fori_loop(0, NP // 16, _zero_deg, 0)

    def _edges(bk, c):
        for j in range(8):
            idx = src_v[bk, pl.ds(16 * j, 16)]
            valid = (128 * bk + 16 * j + iota) < N
            plsc.addupdate_scatter(deg_v, [idx], ones, mask=valid)
        return c
    lax.fori_loop(0, NBLK, _edges, 0)

    pltpu.sync_copy(deg_v, deg_sh.at[sid])
    plsc.subcore_barrier()
    pltpu.sync_copy(deg_sh.at[:, pl.ds(sid * 640, 640)], part_v)

    def _red(q, c):
        acc = part_v[0, pl.ds(16 * q, 16)]
        for t in range(1, NT):
            acc = acc + part_v[t, pl.ds(16 * q, 16)]
        deg_v[pl.ds(16 * q, 16)] = acc
        return c
    lax.fori_loop(0, 40, _red, 0)

    for i in range(40):
        d = deg_v[pl.ds(16 * i, 16)]
        bits = plsc.bitcast(d, _i32)
        y = plsc.bitcast(jnp.int32(0x5F3759DF) - jnp.right_shift(bits, 1), _f32)
        for _ in range(3):
            y = y * (1.5 - 0.5 * d * y * y)
        dv = jnp.where(d > 0.5, y, 0.0)
        dinv_b[pl.ds(16 * i, 16)] = dv
        s1_b[pl.ds(16 * i, 16)] = -(dv * dv)
        s2_b[pl.ds(16 * i, 16)] = -dv
        r_b[pl.ds(16 * i, 16)] = d * dv
    pltpu.sync_copy(dinv_b, dinv_sh.at[pl.ds(sid * 640, 640)])

    @pl.when(cid == 0)
    def _():
        pltpu.sync_copy(dinv_b, dinv_o.at[pl.ds(sid * 640, 640)])
        pltpu.sync_copy(s1_b, s1_o.at[pl.ds(sid * 640, 640)])
        pltpu.sync_copy(s2_b, s2_o.at[pl.ds(sid * 640, 640)])
        pltpu.sync_copy(r_b, r_o.at[pl.ds(sid * 640, 640)])

    plsc.subcore_barrier()

    pltpu.sync_copy(dinv_sh.at[pl.ds(wid * 320, 320)], scl_v)
    for c in range(NCHUNK):
        base = c * NP + wid * 320
        pltpu.sync_copy(zt.at[pl.ds(base, 320), :], buf_v)

        def _scale(i, cr):
            dv = plsc.load_gather(scl_v, [jnp.broadcast_to(i, (16,))])
            for j in range(8):
                buf_v[i, pl.ds(16 * j, 16)] = buf_v[i, pl.ds(16 * j, 16)] * dv
            return cr
        lax.fori_loop(0, 320, _scale, 0)
        pltpu.sync_copy(buf_v, y0.at[pl.ds(base, 320), :])


_prep = functools.partial(
    pl.kernel,
    out_type=(
        jax.ShapeDtypeStruct((NCHUNK * NP, CW), _f32),
        jax.ShapeDtypeStruct((NP,), _f32),
        jax.ShapeDtypeStruct((NP,), _f32),
        jax.ShapeDtypeStruct((NP,), _f32),
        jax.ShapeDtypeStruct((NP,), _f32),
    ),
    mesh=_mesh,
    compiler_params=_sc_params,
    scratch_types=[
        pltpu.VMEM((NP,), _f32),
        pltpu.VMEM((NBLK, 128), _i32),
        pltpu.VMEM((NT, 640), _f32),
        pltpu.VMEM((640,), _f32),
        pltpu.VMEM((640,), _f32),
        pltpu.VMEM((640,), _f32),
        pltpu.VMEM((640,), _f32),
        pltpu.VMEM((320,), _f32),
        pltpu.VMEM((320, CW), _f32),
        pltpu.VMEM_SHARED((NT, NP), _f32),
        pltpu.VMEM_SHARED((NP,), _f32),
    ],
)(_prep_body)


def _spmm_body(y, srcp, dstp, sc, z,
               src_v, dst_v, scs_v, rows_v, acc_sh, sem_a, sem_b):
    cid = lax.axis_index("c")
    sid = lax.axis_index("s")
    z16 = jnp.zeros((16,), _f32)

    pltpu.sync_copy(srcp.at[sid], src_v)
    pltpu.sync_copy(dstp.at[sid], dst_v)
    pltpu.sync_copy(sc.at[pl.ds(sid * 640, 640)], scs_v)

    def _bump(delta):
        def _b(bk, cr):
            for j in range(8):
                src_v[bk, pl.ds(16 * j, 16)] = (
                    src_v[bk, pl.ds(16 * j, 16)] + delta)
            return cr
        lax.fori_loop(0, NBLK, _b, 0)

    _bump(cid * NP)

    for ci in range(NCHUNK // NCORE):
        c = 2 * ci + cid
        if ci > 0:
            _bump(2 * NP)

        def _zob(i, cr):
            for j in range(8):
                rows_v[0, i, pl.ds(16 * j, 16)] = z16
            return cr
        lax.fori_loop(0, 64, _zob, 0)
        for k in range(RPT // 64):
            pltpu.sync_copy(rows_v.at[0],
                            acc_sh.at[pl.ds(sid * RPT + 64 * k, 64), :])
        plsc.subcore_barrier()

        pltpu.async_copy(y.at[src_v.at[0, pl.ds(0, 64)]], rows_v.at[0], sem_a)

        def _pair(p, cr):
            pltpu.async_copy(y.at[src_v.at[p, pl.ds(64, 64)]], rows_v.at[1],
                             sem_b)
            pltpu.make_async_copy(y.at[pl.ds(0, 64), :], rows_v.at[0],
                                  sem_a).wait()
            pltpu.sync_copy(rows_v.at[0], acc_sh.at[dst_v.at[2 * p]],
                            add=True)

            @pl.when(p < EPB // 2 - 1)
            def _():
                pltpu.async_copy(y.at[src_v.at[p + 1, pl.ds(0, 64)]],
                                 rows_v.at[0], sem_a)
            pltpu.make_async_copy(y.at[pl.ds(0, 64), :], rows_v.at[1],
                                  sem_b).wait()
            pltpu.sync_copy(rows_v.at[1], acc_sh.at[dst_v.at[2 * p + 1]],
                            add=True)
            return cr
        lax.fori_loop(0, EPB // 2, _pair, 0)
        plsc.subcore_barrier()

        for k in range(RPT // 64):
            pltpu.sync_copy(acc_sh.at[pl.ds(sid * RPT + 64 * k, 64), :],
                            rows_v.at[0])

            def _scale(i, cr):
                q = 64 * k + i
                sv = plsc.load_gather(scs_v, [jnp.broadcast_to(q, (16,))])
                for j in range(8):
                    rows_v[0, i, pl.ds(16 * j, 16)] = (
                        rows_v[0, i, pl.ds(16 * j, 16)] * sv)
                return cr
            lax.fori_loop(0, 64, _scale, 0)
            pltpu.sync_copy(
                rows_v.at[0], z.at[pl.ds(c * NP + sid * RPT + 64 * k, 64), :])


_spmm = functools.partial(
    pl.kernel,
    out_type=jax.ShapeDtypeStruct((NCHUNK * NP, CW), _f32),
    mesh=_mesh,
    compiler_params=_sc_params,
    scratch_types=[
        pltpu.VMEM((NBLK, 128), _i32),
        pltpu.VMEM((EPB, 64), _i32),
        pltpu.VMEM((640,), _f32),
        pltpu.VMEM((2, 64, CW), _f32),
        pltpu.VMEM_SHARED((NP, CW), _f32),
        pltpu.SemaphoreType.DMA,
        pltpu.SemaphoreType.DMA,
    ],
)(_spmm_body)


def _tc_body(x_ref, y1a_ref, y1b_ref, t2a_ref, t2b_ref, r_ref, w_ref, b_ref,
             o_ref):
    rr = r_ref[...]
    t0 = x_ref[0]
    w0 = w_ref[0]
    w1 = w_ref[1]
    w2 = w_ref[2]
    acc = jnp.dot(t0, w0, preferred_element_type=_f32)
    acc += jnp.dot(y1a_ref[0] * rr, w1[:CW, :], preferred_element_type=_f32)
    acc += jnp.dot(y1b_ref[0] * rr, w1[CW:, :], preferred_element_type=_f32)
    acc += jnp.dot(t2a_ref[0], w2[:CW, :], preferred_element_type=_f32)
    acc += jnp.dot(t2b_ref[0], w2[CW:, :], preferred_element_type=_f32)
    acc += b_ref[...]
    o_ref[0] = jnp.where(acc > 0.0, acc,
                         jnp.exp(jnp.minimum(acc, 0.0)) - 1.0)


def kernel(x, edge_index, W, b):
    Bb, Nn, Fi = x.shape
    Fo = W.shape[2]
    RB = 1000

    ei = edge_index.astype(jnp.int32)
    srcp = jnp.pad(ei[0].reshape(NT, N), ((0, 0), (0, EPT - N))
                   ).reshape(NT, NBLK, 128)
    dstp = jnp.pad(ei[1].reshape(NT, N), ((0, 0), (0, EPT - N)),
                   constant_values=DROW).reshape(NT, EPB, 64)
    zt = x.reshape(Bb, N, 2, CW).transpose(0, 2, 1, 3).reshape(NCHUNK, N, CW)
    zt = jnp.pad(zt, ((0, 0), (0, NP - N), (0, 0))).reshape(NCHUNK * NP, CW)

    y0, dinv2, s12, s22, r2 = _prep(zt, srcp)
    z1 = _spmm(y0, srcp, dstp, s12)
    z2 = _spmm(z1, srcp, dstp, s22)

    Wm = jnp.stack([W[0] - W[2], W[1], 2.0 * W[2]])
    z1v = z1.reshape(NCHUNK, NP, CW)
    z2v = z2.reshape(NCHUNK, NP, CW)
    rv = r2.reshape(NP, 1)

    out = pl.pallas_call(
        _tc_body,
        grid=(Bb, N // RB),
        in_specs=[
            pl.BlockSpec((1, RB, Fi), lambda bb, nn: (bb, nn, 0)),
            pl.BlockSpec((1, RB, CW), lambda bb, nn: (2 * bb, nn, 0)),
            pl.BlockSpec((1, RB, CW), lambda bb, nn: (2 * bb + 1, nn, 0)),
            pl.BlockSpec((1, RB, CW), lambda bb, nn: (2 * bb, nn, 0)),
            pl.BlockSpec((1, RB, CW), lambda bb, nn: (2 * bb + 1, nn, 0)),
            pl.BlockSpec((RB, 1), lambda bb, nn: (nn, 0)),
            pl.BlockSpec((3, Fi, Fo), lambda bb, nn: (0, 0, 0)),
            pl.BlockSpec((1, Fo), lambda bb, nn: (0, 0)),
        ],
        out_specs=pl.BlockSpec((1, RB, Fo), lambda bb, nn: (bb, nn, 0)),
        out_shape=jax.ShapeDtypeStruct((Bb, N, Fo), _f32),
    )(x, z1v, z1v, z2v, z2v, rv, Wm, b.reshape(1, Fo))
    return out
